# baseline (speedup 1.0000x reference)
import functools

import jax
import jax.numpy as jnp
from jax import lax
from jax.experimental import pallas as pl
from jax.experimental.pallas import tpu as pltpu
from jax.experimental.pallas import tpu_sc as plsc

M = 1024
INPUT_DIM = 256
HIDDEN = 64
B = 4
N = 196
T = B * N
TP = 1024

NC = 2
NS = 16
NW = NC * NS
BPW = TP // NW


def _bf16x1_dot(a, b):
    return lax.dot_general(a.astype(jnp.bfloat16), b.astype(jnp.bfloat16),
                           (((1,), (1,)), ((), ())),
                           preferred_element_type=jnp.float32)


def _tc_score_kernel(x_ref, m1_ref, w1_ref, b1_ref, emb_ref, idx_ref):
    X = x_ref[...]
    W1 = w1_ref[...]
    emb = _bf16x1_dot(X, W1)
    emb = jnp.maximum(emb + b1_ref[...], 0.0)
    emb_ref[...] = emb

    mx = jnp.max(emb, axis=1, keepdims=True)
    ex = jnp.exp(emb - mx)
    p = ex / jnp.sum(ex, axis=1, keepdims=True)
    logp = jnp.log(p)

    m1 = m1_ref[...]
    cmx = jnp.max(m1, axis=1, keepdims=True)
    ce = jnp.exp(m1 - cmx)
    m = ce / jnp.sum(ce, axis=1, keepdims=True)
    ent = jnp.sum(m * jnp.log(m), axis=1, keepdims=True)

    crossT = _bf16x1_dot(m, logp)
    klT = ent - crossT

    mn = jnp.min(klT, axis=0, keepdims=True)
    ii = lax.broadcasted_iota(jnp.int32, (M, TP), 0)
    idx = jnp.min(jnp.where(klT == mn, ii, jnp.int32(2**30)),
                  axis=0, keepdims=True)
    idx_ref[...] = idx


def _tc_out_kernel(emb_ref, sel_ref, w2_ref, b2_ref, out_ref):
    emb = emb_ref[...]
    sel = sel_ref[:, 0:HIDDEN]
    w2a = w2_ref[:, 0:HIDDEN]
    w2b = w2_ref[:, HIDDEN:2 * HIDDEN]
    out = _bf16x1_dot(emb, w2a) + _bf16x1_dot(sel, w2b) + b2_ref[...]
    out_ref[...] = out


DPAD = 128


@functools.lru_cache(maxsize=1)
def _make_sc_gather():
    mesh = plsc.VectorSubcoreMesh(core_axis_name="c", subcore_axis_name="s")

    @functools.partial(
        pl.kernel,
        mesh=mesh,
        out_type=jax.ShapeDtypeStruct((TP, DPAD), jnp.float32),
        scratch_types=[
            pltpu.VMEM((BPW,), jnp.int32),
            pltpu.VMEM((BPW, DPAD), jnp.float32),
            pltpu.SemaphoreType.DMA,
        ],
    )
    def _sc_gather(m1_hbm, idx_hbm, out_hbm, idx_v, rows_v, sem):
        wid = lax.axis_index("s") * NC + lax.axis_index("c")
        base = wid * BPW
        pltpu.sync_copy(idx_hbm.at[pl.ds(base, BPW)], idx_v)
        pltpu.async_copy(m1_hbm.at[idx_v], rows_v, sem).wait()
        pltpu.sync_copy(rows_v, out_hbm.at[pl.ds(base, BPW)])

    return _sc_gather


def kernel(input, m1, W1, b1, W2, b2):
    x = jnp.transpose(jnp.squeeze(input, axis=-1), (0, 2, 1)).reshape(T, INPUT_DIM)
    x = jnp.pad(x, ((0, TP - T), (0, 0)))

    emb, idx = pl.pallas_call(
        _tc_score_kernel,
        out_shape=(
            jax.ShapeDtypeStruct((TP, HIDDEN), jnp.float32),
            jax.ShapeDtypeStruct((1, TP), jnp.int32),
        ),
    )(x, m1, W1, b1.reshape(1, HIDDEN))

    m1p = jnp.pad(m1, ((0, 0), (0, DPAD - HIDDEN)))
    sel = _make_sc_gather()(m1p, idx.reshape(TP))

    out_t = pl.pallas_call(
        _tc_out_kernel,
        out_shape=jax.ShapeDtypeStruct((TP, HIDDEN), jnp.float32),
    )(emb, sel, W2, b2.reshape(1, HIDDEN))

    out = jnp.transpose(out_t[:T].reshape(B, N, HIDDEN), (0, 2, 1))
    return out[..., None]


# speedup vs baseline: 1.0265x; 1.0265x over previous
import functools

import jax
import jax.numpy as jnp
from jax import lax
from jax.experimental import pallas as pl
from jax.experimental.pallas import tpu as pltpu
from jax.experimental.pallas import tpu_sc as plsc

M = 1024
INPUT_DIM = 256
HIDDEN = 64
B = 4
N = 196
T = B * N
TP = 1024

NC = 2
NS = 16
NW = NC * NS
BPW = TP // NW

DPAD = 128


def _bf16x1_dot(a, b):
    return lax.dot_general(a.astype(jnp.bfloat16), b.astype(jnp.bfloat16),
                           (((1,), (1,)), ((), ())),
                           preferred_element_type=jnp.float32)


def _tc_score_kernel(x_ref, m1_ref, w1_ref, b1_ref, w2_ref, b2_ref,
                     idx_ref, outa_ref, gt_ref):
    X = x_ref[...]
    W1 = w1_ref[...]
    emb = _bf16x1_dot(X, W1)
    emb = jnp.maximum(emb + b1_ref[...], 0.0)

    mx = jnp.max(emb, axis=1, keepdims=True)
    ex = jnp.exp(emb - mx)
    p = ex / jnp.sum(ex, axis=1, keepdims=True)
    logp = jnp.log(p)

    m1 = m1_ref[...]
    cmx = jnp.max(m1, axis=1, keepdims=True)
    ce = jnp.exp(m1 - cmx)
    m = ce / jnp.sum(ce, axis=1, keepdims=True)
    ent = jnp.sum(m * jnp.log(m), axis=1, keepdims=True)

    crossT = _bf16x1_dot(m, logp)
    klT = ent - crossT

    mn = jnp.min(klT, axis=0, keepdims=True)
    ii = lax.broadcasted_iota(jnp.int32, (M, TP), 0)
    idx_ref[...] = jnp.min(jnp.where(klT == mn, ii, jnp.int32(2**30)),
                           axis=0, keepdims=True)

    w2a = w2_ref[:, 0:HIDDEN]
    w2b = w2_ref[:, HIDDEN:2 * HIDDEN]
    outa_ref[...] = _bf16x1_dot(emb, w2a) + b2_ref[...]

    gt = _bf16x1_dot(m1, w2b)
    gt_ref[...] = jnp.concatenate(
        [gt, jnp.zeros((M, DPAD - HIDDEN), jnp.float32)], axis=1)


@functools.lru_cache(maxsize=1)
def _make_sc_out():
    mesh = plsc.VectorSubcoreMesh(core_axis_name="c", subcore_axis_name="s")

    @functools.partial(
        pl.kernel,
        mesh=mesh,
        out_type=jax.ShapeDtypeStruct((TP, HIDDEN), jnp.float32),
        scratch_types=[
            pltpu.VMEM((BPW,), jnp.int32),
            pltpu.VMEM((BPW, DPAD), jnp.float32),
            pltpu.VMEM((BPW, HIDDEN), jnp.float32),
            pltpu.SemaphoreType.DMA,
        ],
    )
    def _sc_out(gt_hbm, idx_hbm, outa_hbm, out_hbm, idx_v, rows_v, acc_v, sem):
        wid = lax.axis_index("s") * NC + lax.axis_index("c")
        base = wid * BPW
        pltpu.sync_copy(idx_hbm.at[pl.ds(base, BPW)], idx_v)
        pltpu.sync_copy(outa_hbm.at[pl.ds(base, BPW)], acc_v)
        pltpu.async_copy(gt_hbm.at[idx_v], rows_v, sem).wait()

        def body(i, carry):
            for c in range(HIDDEN // 16):
                sl = (i, pl.ds(c * 16, 16))
                acc_v[sl] = acc_v[sl] + rows_v[sl]
            return carry

        lax.fori_loop(0, BPW, body, 0)
        pltpu.sync_copy(acc_v, out_hbm.at[pl.ds(base, BPW)])

    return _sc_out


def kernel(input, m1, W1, b1, W2, b2):
    x = jnp.transpose(jnp.squeeze(input, axis=-1), (0, 2, 1)).reshape(T, INPUT_DIM)
    x = jnp.pad(x, ((0, TP - T), (0, 0)))

    idx, outa, gt = pl.pallas_call(
        _tc_score_kernel,
        out_shape=(
            jax.ShapeDtypeStruct((1, TP), jnp.int32),
            jax.ShapeDtypeStruct((TP, HIDDEN), jnp.float32),
            jax.ShapeDtypeStruct((M, DPAD), jnp.float32),
        ),
    )(x, m1, W1, b1.reshape(1, HIDDEN), W2, b2.reshape(1, HIDDEN))

    out_t = _make_sc_out()(gt, idx.reshape(TP), outa)

    out = jnp.transpose(out_t[:T].reshape(B, N, HIDDEN), (0, 2, 1))
    return out[..., None]


# speedup vs baseline: 1.0406x; 1.0138x over previous
import functools

import jax
import jax.numpy as jnp
from jax import lax
from jax.experimental import pallas as pl
from jax.experimental.pallas import tpu as pltpu
from jax.experimental.pallas import tpu_sc as plsc

M = 1024
INPUT_DIM = 256
HIDDEN = 64
B = 4
N = 196
T = B * N
TP = 1024

NC = 2
NS = 16
NW = NC * NS
BPW = TP // NW

DPAD = 128


def _bf16x1_dot(a, b):
    return lax.dot_general(a.astype(jnp.bfloat16), b.astype(jnp.bfloat16),
                           (((1,), (1,)), ((), ())),
                           preferred_element_type=jnp.float32)


def _tc_score_kernel(x_ref, m1_ref, w1_ref, b1_ref, w2_ref, b2_ref,
                     idx_ref, outa_ref, gt_ref):
    X = x_ref[...]
    W1 = w1_ref[...]
    emb = _bf16x1_dot(X, W1)
    emb = jnp.maximum(emb + b1_ref[...], 0.0)

    mx = jnp.max(emb, axis=1, keepdims=True)
    ex = jnp.exp(emb - mx)
    p = ex / jnp.sum(ex, axis=1, keepdims=True)
    logp = jnp.log(p)

    m1 = m1_ref[...]
    cmx = jnp.max(m1, axis=1, keepdims=True)
    ce = jnp.exp(m1 - cmx)
    m = ce / jnp.sum(ce, axis=1, keepdims=True)
    ent = jnp.sum(m * jnp.log(m), axis=1, keepdims=True)

    crossT = _bf16x1_dot(m, logp)
    klT = ent - crossT

    mn = jnp.min(klT, axis=0, keepdims=True)
    ii = lax.broadcasted_iota(jnp.int32, (M, TP), 0)
    idx_ref[...] = jnp.min(jnp.where(klT == mn, ii, jnp.int32(2**30)),
                           axis=0, keepdims=True)

    w2a = w2_ref[:, 0:HIDDEN]
    w2b = w2_ref[:, HIDDEN:2 * HIDDEN]
    outa_ref[...] = _bf16x1_dot(emb, w2a) + b2_ref[...]

    gt = _bf16x1_dot(m1, w2b)
    gt_ref[...] = jnp.concatenate(
        [gt, jnp.zeros((M, DPAD - HIDDEN), jnp.float32)], axis=1)


@functools.lru_cache(maxsize=1)
def _make_sc_out():
    mesh = plsc.VectorSubcoreMesh(core_axis_name="c", subcore_axis_name="s")

    @functools.partial(
        pl.kernel,
        mesh=mesh,
        out_type=jax.ShapeDtypeStruct((TP, HIDDEN), jnp.float32),
        scratch_types=[
            pltpu.VMEM((BPW,), jnp.int32),
            pltpu.VMEM((BPW, DPAD), jnp.float32),
            pltpu.VMEM((BPW, HIDDEN), jnp.float32),
            pltpu.SemaphoreType.DMA,
            pltpu.SemaphoreType.DMA,
        ],
    )
    def _sc_out(gt_hbm, idx_hbm, outa_hbm, out_hbm, idx_v, rows_v, acc_v,
                sem, sem2):
        wid = lax.axis_index("s") * NC + lax.axis_index("c")
        base = wid * BPW
        outa_cp = pltpu.async_copy(outa_hbm.at[pl.ds(base, BPW)], acc_v, sem2)
        pltpu.sync_copy(idx_hbm.at[pl.ds(base, BPW)], idx_v)
        NSPLIT = 4
        CH = BPW // NSPLIT
        gathers = [
            pltpu.async_copy(gt_hbm.at[idx_v.at[pl.ds(k * CH, CH)]],
                             rows_v.at[pl.ds(k * CH, CH)], sem)
            for k in range(NSPLIT)
        ]
        outa_cp.wait()
        for g in gathers:
            g.wait()

        def body(i, carry):
            for c in range(HIDDEN // 16):
                sl = (i, pl.ds(c * 16, 16))
                acc_v[sl] = acc_v[sl] + rows_v[sl]
            return carry

        lax.fori_loop(0, BPW, body, 0)
        pltpu.sync_copy(acc_v, out_hbm.at[pl.ds(base, BPW)])

    return _sc_out


def kernel(input, m1, W1, b1, W2, b2):
    x = jnp.transpose(jnp.squeeze(input, axis=-1), (0, 2, 1)).reshape(T, INPUT_DIM)
    x = jnp.pad(x, ((0, TP - T), (0, 0)))

    idx, outa, gt = pl.pallas_call(
        _tc_score_kernel,
        out_shape=(
            jax.ShapeDtypeStruct((1, TP), jnp.int32),
            jax.ShapeDtypeStruct((TP, HIDDEN), jnp.float32),
            jax.ShapeDtypeStruct((M, DPAD), jnp.float32),
        ),
    )(x, m1, W1, b1.reshape(1, HIDDEN), W2, b2.reshape(1, HIDDEN))

    out_t = _make_sc_out()(gt, idx.reshape(TP), outa)

    out = jnp.transpose(out_t[:T].reshape(B, N, HIDDEN), (0, 2, 1))
    return out[..., None]


# speedup vs baseline: 1.0646x; 1.0231x over previous
import functools

import jax
import jax.numpy as jnp
from jax import lax
from jax.experimental import pallas as pl
from jax.experimental.pallas import tpu as pltpu
from jax.experimental.pallas import tpu_sc as plsc

M = 1024
INPUT_DIM = 256
HIDDEN = 64
B = 4
N = 196
T = B * N
TP = 1024

NC = 1
NS = 16
NW = NC * NS
BPW = TP // NW

DPAD = 128


def _bf16x1_dot(a, b):
    return lax.dot_general(a.astype(jnp.bfloat16), b.astype(jnp.bfloat16),
                           (((1,), (1,)), ((), ())),
                           preferred_element_type=jnp.float32)


def _tc_score_kernel(x_ref, m1_ref, w1_ref, b1_ref, w2_ref, b2_ref,
                     idx_ref, outa_ref, gt_ref):
    X = x_ref[...]
    W1 = w1_ref[...]
    emb = _bf16x1_dot(X, W1)
    emb = jnp.maximum(emb + b1_ref[...], 0.0)

    mx = jnp.max(emb, axis=1, keepdims=True)
    ex = jnp.exp(emb - mx)
    p = ex / jnp.sum(ex, axis=1, keepdims=True)
    logp = jnp.log(p)

    m1 = m1_ref[...]
    cmx = jnp.max(m1, axis=1, keepdims=True)
    ce = jnp.exp(m1 - cmx)
    m = ce / jnp.sum(ce, axis=1, keepdims=True)
    ent = jnp.sum(m * jnp.log(m), axis=1, keepdims=True)

    crossT = _bf16x1_dot(m, logp)
    klT = ent - crossT

    mn = jnp.min(klT, axis=0, keepdims=True)
    ii = lax.broadcasted_iota(jnp.int32, (M, TP), 0)
    idx_ref[...] = jnp.min(jnp.where(klT == mn, ii, jnp.int32(2**30)),
                           axis=0, keepdims=True)

    w2a = w2_ref[:, 0:HIDDEN]
    w2b = w2_ref[:, HIDDEN:2 * HIDDEN]
    outa_ref[...] = _bf16x1_dot(emb, w2a) + b2_ref[...]

    gt = _bf16x1_dot(m1, w2b)
    gt_ref[...] = jnp.concatenate(
        [gt, jnp.zeros((M, DPAD - HIDDEN), jnp.float32)], axis=1)


@functools.lru_cache(maxsize=1)
def _make_sc_out():
    mesh = plsc.VectorSubcoreMesh(core_axis_name="c", subcore_axis_name="s", num_cores=1)

    @functools.partial(
        pl.kernel,
        mesh=mesh,
        out_type=jax.ShapeDtypeStruct((TP, HIDDEN), jnp.float32),
        scratch_types=[
            pltpu.VMEM((BPW,), jnp.int32),
            pltpu.VMEM((BPW, DPAD), jnp.float32),
            pltpu.VMEM((BPW, HIDDEN), jnp.float32),
            pltpu.SemaphoreType.DMA,
            pltpu.SemaphoreType.DMA,
        ],
    )
    def _sc_out(gt_hbm, idx_hbm, outa_hbm, out_hbm, idx_v, rows_v, acc_v,
                sem, sem2):
        wid = lax.axis_index("s") * NC + lax.axis_index("c")
        base = wid * BPW
        outa_cp = pltpu.async_copy(outa_hbm.at[pl.ds(base, BPW)], acc_v, sem2)
        pltpu.sync_copy(idx_hbm.at[pl.ds(base, BPW)], idx_v)
        NSPLIT = 4
        CH = BPW // NSPLIT
        gathers = [
            pltpu.async_copy(gt_hbm.at[idx_v.at[pl.ds(k * CH, CH)]],
                             rows_v.at[pl.ds(k * CH, CH)], sem)
            for k in range(NSPLIT)
        ]
        outa_cp.wait()
        for g in gathers:
            g.wait()

        def body(i, carry):
            for c in range(HIDDEN // 16):
                sl = (i, pl.ds(c * 16, 16))
                acc_v[sl] = acc_v[sl] + rows_v[sl]
            return carry

        lax.fori_loop(0, BPW, body, 0)
        pltpu.sync_copy(acc_v, out_hbm.at[pl.ds(base, BPW)])

    return _sc_out


def kernel(input, m1, W1, b1, W2, b2):
    x = jnp.transpose(jnp.squeeze(input, axis=-1), (0, 2, 1)).reshape(T, INPUT_DIM)
    x = jnp.pad(x, ((0, TP - T), (0, 0)))

    idx, outa, gt = pl.pallas_call(
        _tc_score_kernel,
        out_shape=(
            jax.ShapeDtypeStruct((1, TP), jnp.int32),
            jax.ShapeDtypeStruct((TP, HIDDEN), jnp.float32),
            jax.ShapeDtypeStruct((M, DPAD), jnp.float32),
        ),
    )(x, m1, W1, b1.reshape(1, HIDDEN), W2, b2.reshape(1, HIDDEN))

    out_t = _make_sc_out()(gt, idx.reshape(TP), outa)

    out = jnp.transpose(out_t[:T].reshape(B, N, HIDDEN), (0, 2, 1))
    return out[..., None]
